# baseline (speedup 1.0000x reference)
import functools

import jax
import jax.numpy as jnp
from jax import lax
from jax.experimental import pallas as pl
from jax.experimental.pallas import tpu as pltpu
from jax.experimental.pallas import tpu_sc as plsc

B = 16384
D = 64
DOUT = D + 4
NC = 2
NS = 16
NW = NC * NS
BPW = B // NW
L = 16
CHUNKS = BPW // L


def _body(idx_hbm, f0_hbm, f1_hbm, f2_hbm, f3_hbm, stats_hbm, table_hbm,
          out_hbm, idx_v, rows_v, feats_v, f4_v, stats_v, gsem):
    wid = lax.axis_index("s") * NC + lax.axis_index("c")
    base = wid * BPW

    pltpu.sync_copy(idx_hbm.at[pl.ds(base, BPW)], idx_v)
    gather = pltpu.async_copy(table_hbm.at[idx_v], rows_v, gsem)

    pltpu.sync_copy(stats_hbm, stats_v)
    for i, f in enumerate((f0_hbm, f1_hbm, f2_hbm, f3_hbm)):
        pltpu.sync_copy(f.at[pl.ds(base, BPW)], feats_v.at[i])

    lane = lax.iota(jnp.int32, L)
    for i in range(4):
        m = plsc.load_gather(stats_v, [jnp.full((L,), 1 + i, jnp.int32)])
        s = plsc.load_gather(stats_v, [jnp.full((L,), 5 + i, jnp.int32)])
        col = jnp.full((L,), i, jnp.int32)
        for c in range(CHUNKS):
            x = feats_v[i, pl.ds(c * L, L)]
            y = (x - m) * s
            plsc.store_scatter(f4_v, [lane + c * L, col], y)

    gather.wait()
    pltpu.sync_copy(rows_v, out_hbm.at[pl.ds(base, BPW), pl.ds(0, D)])
    pltpu.sync_copy(f4_v, out_hbm.at[pl.ds(base, BPW), pl.ds(D, 4)])


def _sc_call(idx, f0, f1, f2, f3, stats, table):
    mesh = plsc.VectorSubcoreMesh(core_axis_name="c", subcore_axis_name="s")
    run = functools.partial(
        pl.kernel,
        mesh=mesh,
        compiler_params=pltpu.CompilerParams(use_tc_tiling_on_sc=False,
                                             needs_layout_passes=False),
        out_type=jax.ShapeDtypeStruct((B, DOUT), jnp.float32),
        scratch_types=[
            pltpu.VMEM((BPW,), jnp.int32),
            pltpu.VMEM((BPW, D), jnp.float32),
            pltpu.VMEM((4, BPW), jnp.float32),
            pltpu.VMEM((BPW, 4), jnp.float32),
            pltpu.VMEM((L,), jnp.float32),
            pltpu.SemaphoreType.DMA,
        ],
    )(_body)
    return run(idx, f0, f1, f2, f3, stats, table)


def kernel(visitorid, user_number_of_views, user_number_of_addtocart,
           user_number_of_purchases, number_of_unique_items,
           table, norm_mean, norm_var):
    idx = visitorid.astype(jnp.int32)
    inv_std = lax.rsqrt(norm_var.astype(jnp.float32) + 1e-7)
    stats = jnp.concatenate(
        [jnp.zeros((1,), jnp.float32), norm_mean.astype(jnp.float32),
         inv_std, jnp.zeros((L - 9,), jnp.float32)])
    return _sc_call(idx, user_number_of_views, user_number_of_addtocart,
                    user_number_of_purchases, number_of_unique_items,
                    stats, table)
